# Initial kernel scaffold; baseline (speedup 1.0000x reference)
import jax
import jax.numpy as jnp
from jax.experimental import pallas as pl


def kernel(x, edge_index, edge_attr, batch, emb, W0, b0, W1, b1, W2, b2, fc1_W, fc1_b, fc2_W, fc2_b, fc3_W, fc3_b):
    raise NotImplementedError("write your pallas kernel here")


# speedup vs baseline: 7.1241x; 7.1241x over previous
import functools

import jax
import jax.numpy as jnp
from jax import lax
from jax.experimental import pallas as pl
from jax.experimental.pallas import tpu as pltpu
from jax.experimental.pallas import tpu_sc as plsc

_N = 10000
_NP = 10240
_E = 160000
_H = 256
_HH = 128
_G = 64

_NC = 2
_NS = 16

_EPW_D = _E // (_NC * _NS)
_DCH = 80
_DNCH = _EPW_D // _DCH
_DTAIL = _EPW_D - _DNCH * _DCH
_RPW = _NP // _NS

_EPW = _E // _NS
_CH = 80
_NCH = _EPW // _CH

_mesh = plsc.VectorSubcoreMesh(core_axis_name="core", subcore_axis_name="subcore")


@functools.partial(
    pl.kernel,
    out_type=jax.ShapeDtypeStruct((2 * _NP, _HH), jnp.float32),
    mesh=_mesh,
    scratch_types=[
        pltpu.VMEM((_DCH,), jnp.int32),
        pltpu.VMEM((_DTAIL,), jnp.int32),
        pltpu.VMEM((_DCH, _HH), jnp.float32),
        pltpu.VMEM_SHARED((_NP, _HH), jnp.float32),
    ],
)
def _sc_deg(dst_hbm, ones_hbm, zeros_hbm, degp,
            idx_v, idxt_v, ones_v, acc_sh):
    c = lax.axis_index("core")
    s = lax.axis_index("subcore")
    r0 = s * _RPW
    pltpu.sync_copy(zeros_hbm, acc_sh.at[pl.ds(r0, _RPW)])
    pltpu.sync_copy(ones_hbm, ones_v)
    plsc.subcore_barrier()

    base = c * (_E // _NC) + s * _EPW_D

    @pl.loop(0, _DNCH)
    def _chunks(i):
        off = base + i * _DCH
        pltpu.sync_copy(dst_hbm.at[pl.ds(off, _DCH)], idx_v)
        pltpu.sync_copy(ones_v, acc_sh.at[idx_v], add=True)

    toff = base + _DNCH * _DCH
    pltpu.sync_copy(dst_hbm.at[pl.ds(toff, _DTAIL)], idxt_v)
    pltpu.sync_copy(ones_v.at[pl.ds(0, _DTAIL)], acc_sh.at[idxt_v], add=True)

    plsc.subcore_barrier()
    pltpu.sync_copy(acc_sh.at[pl.ds(r0, _RPW)],
                    degp.at[pl.ds(c * _NP + r0, _RPW)])


@functools.partial(
    pl.kernel,
    out_type=[jax.ShapeDtypeStruct((_NP, _HH), jnp.float32),
              jax.ShapeDtypeStruct((_NP, _HH), jnp.float32)],
    mesh=_mesh,
    scratch_types=[
        pltpu.VMEM((_CH,), jnp.int32),
        pltpu.VMEM((_CH,), jnp.int32),
        pltpu.VMEM((_CH, _HH), jnp.float32),
        pltpu.VMEM_SHARED((_NP, _HH), jnp.float32),
    ],
)
def _sc_prop(mta_hbm, mtb_hbm, src_hbm, dst_hbm, outa, outb,
             idxs_v, idxd_v, rows_v, acc_sh):
    c = lax.axis_index("core")
    s = lax.axis_index("subcore")
    r0 = s * _RPW
    base = s * _EPW

    def half(tab_hbm, out_hbm):
        pltpu.sync_copy(tab_hbm.at[pl.ds(r0, _RPW)], acc_sh.at[pl.ds(r0, _RPW)])
        plsc.subcore_barrier()

        @pl.loop(0, _NCH)
        def _chunks(i):
            off = base + i * _CH
            pltpu.sync_copy(src_hbm.at[pl.ds(off, _CH)], idxs_v)
            pltpu.sync_copy(tab_hbm.at[idxs_v], rows_v)
            pltpu.sync_copy(dst_hbm.at[pl.ds(off, _CH)], idxd_v)
            pltpu.sync_copy(rows_v, acc_sh.at[idxd_v], add=True)

        plsc.subcore_barrier()
        pltpu.sync_copy(acc_sh.at[pl.ds(r0, _RPW)], out_hbm.at[pl.ds(r0, _RPW)])

    @pl.when(c == 0)
    def _():
        half(mta_hbm, outa)

    @pl.when(c == 1)
    def _():
        half(mtb_hbm, outb)


_RT = 1024
_NT = _NP // _RT
_PREC = lax.Precision.HIGHEST


def _tc_prep_kernel(x_ref, emb_ref, w0_ref, degp_ref,
                    dinv_ref, mta_ref, mtb_ref):
    deg = degp_ref[0, :, 0:1] + degp_ref[1, :, 0:1] + 1.0
    dinv = lax.rsqrt(deg)
    dinv_ref[...] = dinv
    t0 = jnp.dot(emb_ref[...], w0_ref[...],
                 preferred_element_type=jnp.float32, precision=_PREC)
    ids = lax.broadcasted_iota(jnp.int32, (_RT, 128), 1)
    ohw = jnp.where(x_ref[...] == ids, dinv, 0.0)
    mt = jnp.dot(ohw, t0, preferred_element_type=jnp.float32, precision=_PREC)
    mta_ref[...] = mt[:, :_HH]
    mtb_ref[...] = mt[:, _HH:]


def _tc_prep(xc, emb_p, w0, degp3):
    return pl.pallas_call(
        _tc_prep_kernel,
        grid=(_NT,),
        in_specs=[
            pl.BlockSpec((_RT, 1), lambda i: (i, 0)),
            pl.BlockSpec((128, _H), lambda i: (0, 0)),
            pl.BlockSpec((_H, _H), lambda i: (0, 0)),
            pl.BlockSpec((2, _RT, _HH), lambda i: (0, i, 0)),
        ],
        out_specs=[
            pl.BlockSpec((_RT, 1), lambda i: (i, 0)),
            pl.BlockSpec((_RT, _HH), lambda i: (i, 0)),
            pl.BlockSpec((_RT, _HH), lambda i: (i, 0)),
        ],
        out_shape=[jax.ShapeDtypeStruct((_NP, 1), jnp.float32),
                   jax.ShapeDtypeStruct((_NP, _HH), jnp.float32),
                   jax.ShapeDtypeStruct((_NP, _HH), jnp.float32)],
    )(xc, emb_p, w0, degp3)


def _tc_mid_kernel(acca_ref, accb_ref, dinv_ref, b_ref, w_ref,
                   mta_ref, mtb_ref):
    dinv = dinv_ref[...]
    ha = jnp.maximum(acca_ref[...] * dinv + b_ref[:, :_HH], 0.0)
    hb = jnp.maximum(accb_ref[...] * dinv + b_ref[:, _HH:], 0.0)
    mt = (jnp.dot(ha, w_ref[:_HH, :],
                  preferred_element_type=jnp.float32, precision=_PREC)
          + jnp.dot(hb, w_ref[_HH:, :],
                    preferred_element_type=jnp.float32, precision=_PREC))
    mt = mt * dinv
    mta_ref[...] = mt[:, :_HH]
    mtb_ref[...] = mt[:, _HH:]


def _tc_mid(acca, accb, dinv, b2d, w):
    return pl.pallas_call(
        _tc_mid_kernel,
        grid=(_NT,),
        in_specs=[
            pl.BlockSpec((_RT, _HH), lambda i: (i, 0)),
            pl.BlockSpec((_RT, _HH), lambda i: (i, 0)),
            pl.BlockSpec((_RT, 1), lambda i: (i, 0)),
            pl.BlockSpec((1, _H), lambda i: (0, 0)),
            pl.BlockSpec((_H, _H), lambda i: (0, 0)),
        ],
        out_specs=[
            pl.BlockSpec((_RT, _HH), lambda i: (i, 0)),
            pl.BlockSpec((_RT, _HH), lambda i: (i, 0)),
        ],
        out_shape=[jax.ShapeDtypeStruct((_NP, _HH), jnp.float32),
                   jax.ShapeDtypeStruct((_NP, _HH), jnp.float32)],
    )(acca, accb, dinv, b2d, w)


def _tc_final_kernel(acca_ref, accb_ref, dinv_ref, b_ref, batch_ref,
                     fc1w_ref, fc1b_ref, fc2w_ref, fc2b_ref, fc3w_ref,
                     fc3b_ref, out_ref, sums_ref, cnt_ref):
    i = pl.program_id(0)

    @pl.when(i == 0)
    def _():
        sums_ref[...] = jnp.zeros_like(sums_ref)
        cnt_ref[...] = jnp.zeros_like(cnt_ref)

    dinv = dinv_ref[...]
    ha = jnp.maximum(acca_ref[...] * dinv + b_ref[:, :_HH], 0.0)
    hb = jnp.maximum(accb_ref[...] * dinv + b_ref[:, _HH:], 0.0)
    gids = lax.broadcasted_iota(jnp.int32, (_G, _RT), 0)
    m = jnp.where(batch_ref[...] == gids, 1.0, 0.0)
    sums_ref[:, :_HH] += jnp.dot(m, ha, preferred_element_type=jnp.float32,
                                 precision=_PREC)
    sums_ref[:, _HH:] += jnp.dot(m, hb, preferred_element_type=jnp.float32,
                                 precision=_PREC)
    cnt_ref[...] += jnp.sum(m, axis=1, keepdims=True)

    @pl.when(i == _NT - 1)
    def _():
        pooled = sums_ref[...] / jnp.maximum(cnt_ref[...], 1.0)
        z = jnp.maximum(jnp.dot(pooled, fc1w_ref[...],
                                preferred_element_type=jnp.float32,
                                precision=_PREC) + fc1b_ref[...], 0.0)
        z = jnp.maximum(jnp.dot(z, fc2w_ref[...],
                                preferred_element_type=jnp.float32,
                                precision=_PREC) + fc2b_ref[...], 0.0)
        z = jnp.dot(z, fc3w_ref[...], preferred_element_type=jnp.float32,
                    precision=_PREC) + fc3b_ref[...]
        out_ref[...] = z


def _tc_final(acca, accb, dinv, b2d, batch2d, fc1w, fc1b, fc2w, fc2b,
              fc3w, fc3b):
    cst = lambda s: pl.BlockSpec(s, lambda i: tuple(0 for _ in s))
    return pl.pallas_call(
        _tc_final_kernel,
        grid=(_NT,),
        in_specs=[
            pl.BlockSpec((_RT, _HH), lambda i: (i, 0)),
            pl.BlockSpec((_RT, _HH), lambda i: (i, 0)),
            pl.BlockSpec((_RT, 1), lambda i: (i, 0)),
            cst((1, _H)),
            pl.BlockSpec((1, _RT), lambda i: (0, i)),
            cst((_H, _H // 2)),
            cst((1, _H // 2)),
            cst((_H // 2, _H // 4)),
            cst((1, _H // 4)),
            cst((_H // 4, 1)),
            cst((1, 1)),
        ],
        out_specs=cst((_G, 1)),
        out_shape=jax.ShapeDtypeStruct((_G, 1), jnp.float32),
        scratch_shapes=[pltpu.VMEM((_G, _H), jnp.float32),
                        pltpu.VMEM((_G, 1), jnp.float32)],
    )(acca, accb, dinv, b2d, batch2d, fc1w, fc1b, fc2w, fc2b, fc3w, fc3b)


@jax.jit
def _run(x, edge_index, batch, emb, W0, b0, W1, b1, W2, b2,
         fc1_W, fc1_b, fc2_W, fc2_b, fc3_W, fc3_b):
    src = edge_index[0].astype(jnp.int32)
    dst = edge_index[1].astype(jnp.int32)
    xc = jnp.pad(x.astype(jnp.int32).reshape(_N, 1),
                 ((0, _NP - _N), (0, 0)))
    emb_p = jnp.zeros((128, _H), jnp.float32).at[:100, :].set(emb)
    ones_in = jnp.ones((_DCH, _HH), jnp.float32)
    zeros_in = jnp.zeros((_RPW, _HH), jnp.float32)

    degp = _sc_deg(dst, ones_in, zeros_in)
    dinv, mta, mtb = _tc_prep(xc, emb_p, W0, degp.reshape(2, _NP, _HH))
    acca, accb = _sc_prop(mta, mtb, src, dst)
    mta, mtb = _tc_mid(acca, accb, dinv, b0.reshape(1, _H), W1)
    acca, accb = _sc_prop(mta, mtb, src, dst)
    mta, mtb = _tc_mid(acca, accb, dinv, b1.reshape(1, _H), W2)
    acca, accb = _sc_prop(mta, mtb, src, dst)
    z = _tc_final(acca, accb, dinv, b2.reshape(1, _H),
                  jnp.pad(batch.astype(jnp.int32).reshape(1, _N),
                          ((0, 0), (0, _NP - _N)), constant_values=_G),
                  fc1_W, fc1_b.reshape(1, _H // 2),
                  fc2_W, fc2_b.reshape(1, _H // 4),
                  fc3_W, fc3_b.reshape(1, 1))
    return z.reshape(_G)


def kernel(x, edge_index, edge_attr, batch, emb, W0, b0, W1, b1, W2, b2,
           fc1_W, fc1_b, fc2_W, fc2_b, fc3_W, fc3_b):
    del edge_attr
    return _run(x, edge_index, batch, emb, W0, b0, W1, b1, W2, b2,
                fc1_W, fc1_b, fc2_W, fc2_b, fc3_W, fc3_b)


# speedup vs baseline: 12.0369x; 1.6896x over previous
import functools

import jax
import jax.numpy as jnp
from jax import lax
from jax.experimental import pallas as pl
from jax.experimental.pallas import tpu as pltpu
from jax.experimental.pallas import tpu_sc as plsc

_N = 10000
_NP = 10240
_E = 160000
_H = 256
_HH = 128
_G = 64

_NC = 2
_NS = 16

_EPW_D = _E // (_NC * _NS)
_DCH = 80
_DNCH = _EPW_D // _DCH
_DTAIL = _EPW_D - _DNCH * _DCH
_RPW = _NP // _NS

_CH = 128
_NCHT = _E // _CH
_CPS = _NCHT // _NS
_NPAIR = _CPS // 2

_mesh = plsc.VectorSubcoreMesh(core_axis_name="core", subcore_axis_name="subcore")


@functools.partial(
    pl.kernel,
    out_type=jax.ShapeDtypeStruct((2 * _NP, _HH), jnp.float32),
    mesh=_mesh,
    scratch_types=[
        pltpu.VMEM((_DCH,), jnp.int32),
        pltpu.VMEM((_DTAIL,), jnp.int32),
        pltpu.VMEM((_DCH, _HH), jnp.float32),
        pltpu.VMEM_SHARED((_NP, _HH), jnp.float32),
    ],
)
def _sc_deg(dst_hbm, ones_hbm, zeros_hbm, degp,
            idx_v, idxt_v, ones_v, acc_sh):
    c = lax.axis_index("core")
    s = lax.axis_index("subcore")
    r0 = s * _RPW
    pltpu.sync_copy(zeros_hbm, acc_sh.at[pl.ds(r0, _RPW)])
    pltpu.sync_copy(ones_hbm, ones_v)
    plsc.subcore_barrier()

    base = c * (_E // _NC) + s * _EPW_D

    @pl.loop(0, _DNCH)
    def _chunks(i):
        off = base + i * _DCH
        pltpu.sync_copy(dst_hbm.at[pl.ds(off, _DCH)], idx_v)
        pltpu.sync_copy(ones_v, acc_sh.at[idx_v], add=True)

    toff = base + _DNCH * _DCH
    pltpu.sync_copy(dst_hbm.at[pl.ds(toff, _DTAIL)], idxt_v)
    pltpu.sync_copy(ones_v.at[pl.ds(0, _DTAIL)], acc_sh.at[idxt_v], add=True)

    plsc.subcore_barrier()
    pltpu.sync_copy(acc_sh.at[pl.ds(r0, _RPW)],
                    degp.at[pl.ds(c * _NP + r0, _RPW)])


@functools.partial(
    pl.kernel,
    out_type=[jax.ShapeDtypeStruct((_NP, _HH), jnp.float32),
              jax.ShapeDtypeStruct((_NP, _HH), jnp.float32)],
    mesh=_mesh,
    scratch_types=[
        pltpu.VMEM((2, _CH), jnp.int32),
        pltpu.VMEM((2, _CH), jnp.int32),
        pltpu.VMEM((_CH, _HH), jnp.float32),
        pltpu.VMEM((_CH, _HH), jnp.float32),
        pltpu.VMEM_SHARED((_NP, _HH), jnp.float32),
        pltpu.SemaphoreType.DMA,
        pltpu.SemaphoreType.DMA,
    ],
)
def _sc_prop(mta_hbm, mtb_hbm, eidx_hbm, outa, outb,
             e0_v, e1_v, rows0_v, rows1_v, acc_sh, gsem0, gsem1):
    c = lax.axis_index("core")
    s = lax.axis_index("subcore")
    r0 = s * _RPW
    bc = _CPS * s + jnp.minimum(s, 2)

    def half(tab_hbm, out_hbm):
        pltpu.sync_copy(tab_hbm.at[pl.ds(r0, _RPW)], acc_sh.at[pl.ds(r0, _RPW)])
        plsc.subcore_barrier()

        def load_idx(k, ebuf):
            pltpu.sync_copy(eidx_hbm.at[:, pl.ds((bc + k) * _CH, _CH)], ebuf)

        def gather_start(ebuf, rbuf, sem):
            pltpu.async_copy(tab_hbm.at[ebuf.at[0]], rbuf, sem)

        def gather_wait(ebuf, rbuf, sem):
            pltpu.make_async_copy(tab_hbm.at[ebuf.at[0]], rbuf, sem).wait()

        def scatter(ebuf, rbuf):
            pltpu.sync_copy(rbuf, acc_sh.at[ebuf.at[1]], add=True)

        load_idx(0, e0_v)
        gather_start(e0_v, rows0_v, gsem0)

        @pl.loop(0, _NPAIR)
        def _pair(j):
            load_idx(2 * j + 1, e1_v)
            gather_start(e1_v, rows1_v, gsem1)
            gather_wait(e0_v, rows0_v, gsem0)
            scatter(e0_v, rows0_v)

            @pl.when(j < _NPAIR - 1)
            def _():
                load_idx(2 * j + 2, e0_v)
                gather_start(e0_v, rows0_v, gsem0)

            gather_wait(e1_v, rows1_v, gsem1)
            scatter(e1_v, rows1_v)

        @pl.when(s < 2)
        def _():
            load_idx(_CPS, e0_v)
            pltpu.sync_copy(tab_hbm.at[e0_v.at[0]], rows0_v)
            scatter(e0_v, rows0_v)

        plsc.subcore_barrier()
        pltpu.sync_copy(acc_sh.at[pl.ds(r0, _RPW)], out_hbm.at[pl.ds(r0, _RPW)])

    @pl.when(c == 0)
    def _():
        half(mta_hbm, outa)

    @pl.when(c == 1)
    def _():
        half(mtb_hbm, outb)


_RT = 1024
_NT = _NP // _RT
_PREC = lax.Precision.HIGHEST


def _tc_prep_kernel(x_ref, emb_ref, w0_ref, degp_ref,
                    dinv_ref, mta_ref, mtb_ref):
    deg = degp_ref[0, :, 0:1] + degp_ref[1, :, 0:1] + 1.0
    dinv = lax.rsqrt(deg)
    dinv_ref[...] = dinv
    ids = lax.broadcasted_iota(jnp.int32, (_RT, 128), 1)
    oh = jnp.where(x_ref[...] == ids, 1.0, 0.0)
    h0 = jnp.dot(oh, emb_ref[...], preferred_element_type=jnp.float32)
    m0 = jnp.dot(h0, w0_ref[...], preferred_element_type=jnp.float32)
    mt = m0 * dinv
    mta_ref[...] = mt[:, :_HH]
    mtb_ref[...] = mt[:, _HH:]


def _tc_prep(xc, emb_p, w0, degp3):
    return pl.pallas_call(
        _tc_prep_kernel,
        grid=(_NT,),
        in_specs=[
            pl.BlockSpec((_RT, 1), lambda i: (i, 0)),
            pl.BlockSpec((128, _H), lambda i: (0, 0)),
            pl.BlockSpec((_H, _H), lambda i: (0, 0)),
            pl.BlockSpec((2, _RT, _HH), lambda i: (0, i, 0)),
        ],
        out_specs=[
            pl.BlockSpec((_RT, 1), lambda i: (i, 0)),
            pl.BlockSpec((_RT, _HH), lambda i: (i, 0)),
            pl.BlockSpec((_RT, _HH), lambda i: (i, 0)),
        ],
        out_shape=[jax.ShapeDtypeStruct((_NP, 1), jnp.float32),
                   jax.ShapeDtypeStruct((_NP, _HH), jnp.float32),
                   jax.ShapeDtypeStruct((_NP, _HH), jnp.float32)],
    )(xc, emb_p, w0, degp3)


def _tc_mid_kernel(acca_ref, accb_ref, dinv_ref, b_ref, w_ref,
                   mta_ref, mtb_ref):
    dinv = dinv_ref[...]
    h = jnp.maximum(
        jnp.concatenate([acca_ref[...], accb_ref[...]], axis=1) * dinv
        + b_ref[...], 0.0)
    mt = jnp.dot(h, w_ref[...], preferred_element_type=jnp.float32) * dinv
    mta_ref[...] = mt[:, :_HH]
    mtb_ref[...] = mt[:, _HH:]


def _tc_mid(acca, accb, dinv, b2d, w):
    return pl.pallas_call(
        _tc_mid_kernel,
        grid=(_NT,),
        in_specs=[
            pl.BlockSpec((_RT, _HH), lambda i: (i, 0)),
            pl.BlockSpec((_RT, _HH), lambda i: (i, 0)),
            pl.BlockSpec((_RT, 1), lambda i: (i, 0)),
            pl.BlockSpec((1, _H), lambda i: (0, 0)),
            pl.BlockSpec((_H, _H), lambda i: (0, 0)),
        ],
        out_specs=[
            pl.BlockSpec((_RT, _HH), lambda i: (i, 0)),
            pl.BlockSpec((_RT, _HH), lambda i: (i, 0)),
        ],
        out_shape=[jax.ShapeDtypeStruct((_NP, _HH), jnp.float32),
                   jax.ShapeDtypeStruct((_NP, _HH), jnp.float32)],
    )(acca, accb, dinv, b2d, w)


def _tc_h3_kernel(acca_ref, accb_ref, dinv_ref, b_ref, h_ref):
    h_ref[...] = jnp.maximum(
        jnp.concatenate([acca_ref[...], accb_ref[...]], axis=1) * dinv_ref[...]
        + b_ref[...], 0.0)


def _tc_h3(acca, accb, dinv, b2d):
    return pl.pallas_call(
        _tc_h3_kernel,
        grid=(_NT,),
        in_specs=[
            pl.BlockSpec((_RT, _HH), lambda i: (i, 0)),
            pl.BlockSpec((_RT, _HH), lambda i: (i, 0)),
            pl.BlockSpec((_RT, 1), lambda i: (i, 0)),
            pl.BlockSpec((1, _H), lambda i: (0, 0)),
        ],
        out_specs=pl.BlockSpec((_RT, _H), lambda i: (i, 0)),
        out_shape=jax.ShapeDtypeStruct((_NP, _H), jnp.float32),
    )(acca, accb, dinv, b2d)


@jax.jit
def _run(x, edge_index, batch, emb, W0, b0, W1, b1, W2, b2,
         fc1_W, fc1_b, fc2_W, fc2_b, fc3_W, fc3_b):
    eidx = edge_index.astype(jnp.int32)
    dst = eidx[1]
    xc = jnp.pad(x.astype(jnp.int32).reshape(_N, 1),
                 ((0, _NP - _N), (0, 0)))
    emb_p = jnp.zeros((128, _H), jnp.float32).at[:100, :].set(emb)
    ones_in = jnp.ones((_DCH, _HH), jnp.float32)
    zeros_in = jnp.zeros((_RPW, _HH), jnp.float32)

    degp = _sc_deg(dst, ones_in, zeros_in)
    dinv, mta, mtb = _tc_prep(xc, emb_p, W0, degp.reshape(2, _NP, _HH))
    acca, accb = _sc_prop(mta, mtb, eidx)
    mta, mtb = _tc_mid(acca, accb, dinv, b0.reshape(1, _H), W1)
    acca, accb = _sc_prop(mta, mtb, eidx)
    mta, mtb = _tc_mid(acca, accb, dinv, b1.reshape(1, _H), W2)
    acca, accb = _sc_prop(mta, mtb, eidx)
    h3 = _tc_h3(acca, accb, dinv, b2.reshape(1, _H))[:_N]
    sums = jax.ops.segment_sum(h3, batch, num_segments=_G)
    counts = jax.ops.segment_sum(jnp.ones((_N,), h3.dtype), batch,
                                 num_segments=_G)
    pooled = sums / jnp.maximum(counts, 1.0)[:, None]
    z = jax.nn.relu(pooled @ fc1_W + fc1_b)
    z = jax.nn.relu(z @ fc2_W + fc2_b)
    z = z @ fc3_W + fc3_b
    z = z.squeeze(-1)
    return z


def kernel(x, edge_index, edge_attr, batch, emb, W0, b0, W1, b1, W2, b2,
           fc1_W, fc1_b, fc2_W, fc2_b, fc3_W, fc3_b):
    del edge_attr
    return _run(x, edge_index, batch, emb, W0, b0, W1, b1, W2, b2,
                fc1_W, fc1_b, fc2_W, fc2_b, fc3_W, fc3_b)
